# baseline (speedup 1.0000x reference)
import functools

import jax
import jax.numpy as jnp
from jax import lax
from jax.experimental import pallas as pl
from jax.experimental.pallas import tpu as pltpu
from jax.experimental.pallas import tpu_sc as plsc

_NC = 2
_NS = 16
_NW = _NC * _NS
_CHUNK = 128
_EPS = 1e-5


def _pick_group(ch, dout):
    budget = 300 * 1024
    for k in (10, 8, 5, 4, 2, 1):
        if ch % k == 0 and 2 * k * _CHUNK * dout * 4 <= budget:
            return k
    return 1


def _make_sc_scatter(n_pad, dout, ch):
    r = n_pad // _NS
    k = _pick_group(ch, dout)
    ng = ch // k
    mesh = plsc.VectorSubcoreMesh(core_axis_name="c", subcore_axis_name="s")

    @functools.partial(
        pl.kernel,
        out_type=jax.ShapeDtypeStruct((_NC * n_pad, dout), jnp.float32),
        mesh=mesh,
        compiler_params=pltpu.CompilerParams(use_tc_tiling_on_sc=False),
        scratch_types=[
            pltpu.VMEM_SHARED((n_pad, dout), jnp.float32),
            pltpu.VMEM((ch, _CHUNK), jnp.int32),
            pltpu.VMEM((ch, _CHUNK), jnp.int32),
            pltpu.VMEM((2, k, _CHUNK, dout), jnp.float32),
            pltpu.SemaphoreType.DMA,
            pltpu.SemaphoreType.DMA,
        ],
    )
    def sc_kernel(hs_hbm, src_hbm, dst_hbm, out_hbm, acc_sh, src_v, dst_v,
                  rows_v, sem_g, sem_s):
        c = lax.axis_index("c")
        s = lax.axis_index("s")
        w = c * _NS + s
        pltpu.sync_copy(hs_hbm.at[pl.ds(s * r, r)], acc_sh.at[pl.ds(s * r, r)])
        pltpu.sync_copy(src_hbm.at[w], src_v)
        pltpu.sync_copy(dst_hbm.at[w], dst_v)
        plsc.subcore_barrier()

        def issue(t, p):
            for b in range(k):
                pltpu.async_copy(hs_hbm.at[src_v.at[t * k + b]],
                                 rows_v.at[p, b], sem_g)

        def drain_g(p):
            for b in range(k):
                pltpu.make_async_copy(hs_hbm.at[pl.ds(0, _CHUNK)],
                                      rows_v.at[p, b], sem_g).wait()

        def scat(t, p):
            for b in range(k):
                pltpu.async_copy(rows_v.at[p, b],
                                 acc_sh.at[dst_v.at[t * k + b]], sem_s,
                                 add=True)

        def drain_s():
            for b in range(k):
                pltpu.make_async_copy(hs_hbm.at[pl.ds(0, _CHUNK)],
                                      rows_v.at[0, b], sem_s).wait()

        issue(0, 0)
        drain_g(0)
        scat(0, 0)
        if ng > 1:
            issue(1, 1)

            def steady(t, carry):
                p = lax.rem(t, 2)
                drain_g(p)
                scat(t, p)
                drain_s()
                issue(t + 1, 1 - p)
                return carry

            if ng > 2:
                lax.fori_loop(1, ng - 1, steady, 0)
            pl_ = (ng - 1) % 2
            drain_g(pl_)
            scat(ng - 1, pl_)
        for _ in range(min(ng, 2)):
            drain_s()
        plsc.subcore_barrier()
        pltpu.sync_copy(acc_sh.at[pl.ds(s * r, r)],
                        out_hbm.at[pl.ds(c * n_pad + s * r, r)])

    return sc_kernel


def _make_sc_degree(n_pad, dout, ch):
    r = n_pad // _NS
    mesh = plsc.VectorSubcoreMesh(core_axis_name="c", subcore_axis_name="s")

    @functools.partial(
        pl.kernel,
        out_type=jax.ShapeDtypeStruct((_NC * n_pad, dout), jnp.float32),
        mesh=mesh,
        compiler_params=pltpu.CompilerParams(use_tc_tiling_on_sc=False),
        scratch_types=[
            pltpu.VMEM_SHARED((n_pad, dout), jnp.float32),
            pltpu.VMEM((ch, _CHUNK), jnp.int32),
            pltpu.VMEM((_CHUNK, dout), jnp.float32),
            pltpu.SemaphoreType.DMA,
        ],
    )
    def sc_kernel(ones_hbm, dst_hbm, out_hbm, acc_sh, dst_v, rows_v, sem_s):
        c = lax.axis_index("c")
        s = lax.axis_index("s")
        w = c * _NS + s
        pltpu.sync_copy(ones_hbm.at[pl.ds(s * r, r)],
                        acc_sh.at[pl.ds(s * r, r)])
        pltpu.sync_copy(dst_hbm.at[w], dst_v)
        pltpu.sync_copy(ones_hbm.at[pl.ds(0, _CHUNK)], rows_v)
        plsc.subcore_barrier()

        def body(j, carry):
            pltpu.async_copy(rows_v, acc_sh.at[dst_v.at[j]], sem_s, add=True)
            return carry

        lax.fori_loop(0, ch, body, 0)

        def drain(j, carry):
            pltpu.make_async_copy(ones_hbm.at[pl.ds(0, _CHUNK)], rows_v,
                                  sem_s).wait()
            return carry

        lax.fori_loop(0, ch, drain, 0)
        plsc.subcore_barrier()
        pltpu.sync_copy(acc_sh.at[pl.ds(s * r, r)],
                        out_hbm.at[pl.ds(c * n_pad + s * r, r)])

    return sc_kernel


def _row_mask(n, n_pad):
    return (lax.broadcasted_iota(jnp.int32, (n_pad, 1), 0) < n).astype(
        jnp.float32)


def _bn_relu(z, n, n_pad, g, be):
    mask = _row_mask(n, n_pad)
    zm = z * mask
    mu = jnp.sum(zm, axis=0, keepdims=True) / n
    d = (z - mu) * mask
    var = jnp.sum(d * d, axis=0, keepdims=True) / n
    y = (z - mu) * lax.rsqrt(var + _EPS) * g + be
    return jnp.maximum(y, 0.0) * mask


def _tc_mm1_body(n, x_ref, w_ref, h_ref):
    h_ref[:n] = jnp.dot(x_ref[...], w_ref[...],
                        preferred_element_type=jnp.float32)
    h_ref[n:] = jnp.zeros((h_ref.shape[0] - n, h_ref.shape[1]), jnp.float32)


def _tc_scale_body(h_ref, deg_ref, dis_ref, hs_ref):
    n_pad = h_ref.shape[0]
    deg = deg_ref[:n_pad, 0:1] + deg_ref[n_pad:, 0:1] - 1.0
    dis = lax.rsqrt(deg)
    dis_ref[...] = dis
    hs_ref[...] = h_ref[...] * dis


def _tc_post1_body(n, acc_ref, hs_ref, dis_ref, g_ref, be_ref, out_ref):
    n_pad = hs_ref.shape[0]
    dis = dis_ref[...]
    agg = acc_ref[:n_pad] + acc_ref[n_pad:] - hs_ref[...]
    y = _bn_relu(agg * dis, n, n_pad, g_ref[...], be_ref[...])
    out_ref[...] = y * dis


def _tc_mid_body(n, acc_ref, hs_ref, dis_ref, g_ref, be_ref, w_ref, out_ref):
    n_pad = hs_ref.shape[0]
    dis = dis_ref[...]
    agg = acc_ref[:n_pad] + acc_ref[n_pad:] - hs_ref[...]
    u = jnp.dot(agg * dis, w_ref[...], preferred_element_type=jnp.float32)
    y = _bn_relu(u, n, n_pad, g_ref[...], be_ref[...])
    out_ref[...] = y * dis


def _tc_final_body(n, acc_ref, hs_ref, dis_ref, g_ref, be_ref, w_ref, wo_ref,
                   bo_ref, out_ref):
    n_pad = hs_ref.shape[0]
    agg = acc_ref[:n_pad] + acc_ref[n_pad:] - hs_ref[...]
    u = jnp.dot(agg * dis_ref[...], w_ref[...],
                preferred_element_type=jnp.float32)
    y = _bn_relu(u, n, n_pad, g_ref[...], be_ref[...])
    pooled = jnp.sum(y, axis=0, keepdims=True) / n
    out_ref[...] = jnp.dot(
        pooled, wo_ref[...], preferred_element_type=jnp.float32) + bo_ref[...]


def _tc_mm1(n, n_pad, x, w1):
    return pl.pallas_call(
        functools.partial(_tc_mm1_body, n),
        out_shape=jax.ShapeDtypeStruct((n_pad, w1.shape[1]), jnp.float32),
    )(x, w1)


def _tc_scale(n_pad, h1, deg_acc):
    return pl.pallas_call(
        _tc_scale_body,
        out_shape=(
            jax.ShapeDtypeStruct((n_pad, 1), jnp.float32),
            jax.ShapeDtypeStruct((n_pad, h1.shape[1]), jnp.float32),
        ),
    )(h1, deg_acc)


def _tc_post1(n, n_pad, acc, hs, dis, g, be):
    return pl.pallas_call(
        functools.partial(_tc_post1_body, n),
        out_shape=jax.ShapeDtypeStruct((n_pad, hs.shape[1]), jnp.float32),
    )(acc, hs, dis, g, be)


def _tc_mid(n, n_pad, acc, hs, dis, g, be, w):
    return pl.pallas_call(
        functools.partial(_tc_mid_body, n),
        out_shape=jax.ShapeDtypeStruct((n_pad, w.shape[1]), jnp.float32),
    )(acc, hs, dis, g, be, w)


def _tc_final(n, acc, hs, dis, g, be, w, wo, bo):
    return pl.pallas_call(
        functools.partial(_tc_final_body, n),
        out_shape=jax.ShapeDtypeStruct((1, wo.shape[1]), jnp.float32),
    )(acc, hs, dis, g, be, w, wo, bo)


def kernel(x, edge_index, W1, b1, g1, be1, W2, b2, g2, be2, W3, b3, g3, be3,
           W4, b4, g4, be4, Wo, bo):
    n, f = x.shape
    e = edge_index.shape[1]
    n_pad = ((n + 1 + 255) // 256) * 256
    ch = -(-e // (_NW * _CHUNK))
    ep = _NW * ch * _CHUNK

    pad = jnp.full((ep - e,), n, dtype=jnp.int32)
    src = jnp.concatenate([edge_index[0], pad]).reshape(_NW, ch, _CHUNK)
    dst = jnp.concatenate([edge_index[1], pad]).reshape(_NW, ch, _CHUNK)
    ones = jnp.ones((n_pad, 8), jnp.float32)
    deg_acc = _make_sc_degree(n_pad, 8, ch)(ones, dst)
    h1 = _tc_mm1(n, n_pad, x, W1)

    dis, hs1 = _tc_scale(n_pad, h1, deg_acc)
    sc8 = _make_sc_scatter(n_pad, 8, ch)
    acc1 = sc8(hs1, src, dst)
    ys1 = _tc_post1(n, n_pad, acc1, hs1, dis, g1, be1)
    acc2 = sc8(ys1, src, dst)
    ys2 = _tc_mid(n, n_pad, acc2, ys1, dis, g2, be2, W2)
    acc3 = _make_sc_scatter(n_pad, 16, ch)(ys2, src, dst)
    ys3 = _tc_mid(n, n_pad, acc3, ys2, dis, g3, be3, W3)
    acc4 = _make_sc_scatter(n_pad, 32, ch)(ys3, src, dst)
    return _tc_final(n, acc4, ys3, dis, g4, be4, W4, Wo, bo)
